# baseline (speedup 1.0000x reference)
import functools

import jax
import jax.numpy as jnp
from jax import lax
from jax.experimental import pallas as pl
from jax.experimental.pallas import tpu as pltpu
from jax.experimental.pallas import tpu_sc as plsc

R, C, K = 16384, 512, 128
NC, NS = 2, 16
NW = NC * NS
ROWS_PER_W = R // NW
CH = 64
NCH = ROWS_PER_W // CH

_mesh = plsc.VectorSubcoreMesh(core_axis_name="c", subcore_axis_name="s")


@functools.partial(
    pl.kernel,
    mesh=_mesh,
    out_type=jax.ShapeDtypeStruct((R, K), jnp.float32),
    scratch_types=[
        pltpu.VMEM((2, CH, C), jnp.float32),
        pltpu.VMEM((2, CH, K), jnp.float32),
        pltpu.SemaphoreType.DMA,
        pltpu.SemaphoreType.DMA,
    ],
    compiler_params=pltpu.CompilerParams(needs_layout_passes=False),
)
def _sc_extract(in_hbm, out_hbm, inbuf, outbuf, in_sem, out_sem):
    wid = lax.axis_index("s") * NC + lax.axis_index("c")
    lane = lax.iota(jnp.int32, 16)
    colv = [lane * 4 + 64 * g for g in range(K // 16)]
    base = wid * ROWS_PER_W

    def start_in(ch):
        return pltpu.async_copy(
            in_hbm.at[pl.ds(base + ch * CH, CH), :], inbuf.at[ch % 2], in_sem)

    def start_out(ch):
        return pltpu.async_copy(
            outbuf.at[ch % 2], out_hbm.at[pl.ds(base + ch * CH, CH), :], out_sem)

    in_copies = {0: start_in(0)}
    out_copies = {}
    for ch in range(NCH):
        if ch + 1 < NCH:
            in_copies[ch + 1] = start_in(ch + 1)
        in_copies.pop(ch).wait()
        if ch >= 2:
            out_copies.pop(ch - 2).wait()

        ib = inbuf.at[ch % 2]
        ob = outbuf.at[ch % 2]

        def body(r, _, ib=ib, ob=ob, colv=colv):
            rows = jnp.full((16,), r, jnp.int32)
            for g in range(K // 16):
                ob[r, pl.ds(g * 16, 16)] = plsc.load_gather(ib, [rows, colv[g]])
            return 0

        lax.fori_loop(0, CH, body, 0, unroll=8)
        out_copies[ch] = start_out(ch)

    for ch in sorted(out_copies):
        out_copies.pop(ch).wait()


def kernel(inputs):
    return _sc_extract(inputs)


# speedup vs baseline: 1.3328x; 1.3328x over previous
import functools

import jax
import jax.numpy as jnp
from jax import lax
from jax.experimental import pallas as pl
from jax.experimental.pallas import tpu as pltpu
from jax.experimental.pallas import tpu_sc as plsc

R, C, K = 16384, 512, 128
NC, NS = 2, 16
NW = NC * NS
ROWS_PER_W = R // NW
CH = 64
NCH = ROWS_PER_W // CH

_mesh = plsc.VectorSubcoreMesh(core_axis_name="c", subcore_axis_name="s")


@functools.partial(
    pl.kernel,
    mesh=_mesh,
    out_type=jax.ShapeDtypeStruct((R, K), jnp.float32),
    scratch_types=[
        pltpu.VMEM((2, CH, C), jnp.float32),
        pltpu.VMEM((2, CH, K), jnp.float32),
        pltpu.SemaphoreType.DMA,
        pltpu.SemaphoreType.DMA,
    ],
    compiler_params=pltpu.CompilerParams(needs_layout_passes=False),
)
def _sc_extract(in_hbm, out_hbm, inbuf, outbuf, in_sem, out_sem):
    wid = lax.axis_index("s") * NC + lax.axis_index("c")
    lane = lax.iota(jnp.int32, 16)
    colv = [lane * 4 + 64 * g for g in range(K // 16)]
    base = wid * ROWS_PER_W

    def in_copy(ch):
        return pltpu.make_async_copy(
            in_hbm.at[pl.ds(base + ch * CH, CH), :],
            inbuf.at[lax.rem(ch, 2)], in_sem)

    def out_copy(ch):
        return pltpu.make_async_copy(
            outbuf.at[lax.rem(ch, 2)],
            out_hbm.at[pl.ds(base + ch * CH, CH), :], out_sem)

    in_copy(0).start()
    in_copy(1).start()

    def chunk_body(ch, _):
        in_copy(ch).wait()

        @pl.when(ch >= 2)
        def _():
            out_copy(ch - 2).wait()

        ib = inbuf.at[lax.rem(ch, 2)]
        ob = outbuf.at[lax.rem(ch, 2)]

        def body(r, _, ib=ib, ob=ob, colv=colv):
            rows = jnp.full((16,), r, jnp.int32)
            for g in range(K // 16):
                ob[r, pl.ds(g * 16, 16)] = plsc.load_gather(ib, [rows, colv[g]])
            return 0

        lax.fori_loop(0, CH, body, 0, unroll=1)
        out_copy(ch).start()

        @pl.when(ch + 2 < NCH)
        def _():
            in_copy(ch + 2).start()

        return 0

    lax.fori_loop(0, NCH, chunk_body, 0, unroll=1)
    out_copy(NCH - 2).wait()
    out_copy(NCH - 1).wait()


def kernel(inputs):
    return _sc_extract(inputs)


# speedup vs baseline: 1.5441x; 1.1585x over previous
import functools

import jax
import jax.numpy as jnp
from jax import lax
from jax.experimental import pallas as pl
from jax.experimental.pallas import tpu as pltpu
from jax.experimental.pallas import tpu_sc as plsc

R, C, K = 16384, 512, 128
NC, NS = 2, 16
NW = NC * NS
ROWS_PER_W = R // NW
CH = 64
NCH = ROWS_PER_W // CH

_mesh = plsc.VectorSubcoreMesh(core_axis_name="c", subcore_axis_name="s")


@functools.partial(
    pl.kernel,
    mesh=_mesh,
    out_type=jax.ShapeDtypeStruct((R, K), jnp.float32),
    scratch_types=[
        pltpu.VMEM((2, CH, C), jnp.float32),
        pltpu.VMEM((2, CH, K), jnp.float32),
        pltpu.SemaphoreType.DMA,
        pltpu.SemaphoreType.DMA,
    ],
    compiler_params=pltpu.CompilerParams(needs_layout_passes=False),
)
def _sc_extract(in_hbm, out_hbm, inbuf, outbuf, in_sem, out_sem):
    wid = lax.axis_index("s") * NC + lax.axis_index("c")
    lane = lax.iota(jnp.int32, 16)
    colv = [lane * 4 + 64 * g for g in range(K // 16)]
    base = wid * ROWS_PER_W

    def in_copy(ch):
        return pltpu.make_async_copy(
            in_hbm.at[pl.ds(base + ch * CH, CH), :],
            inbuf.at[lax.rem(ch, 2)], in_sem)

    def out_copy(ch):
        return pltpu.make_async_copy(
            outbuf.at[lax.rem(ch, 2)],
            out_hbm.at[pl.ds(base + ch * CH, CH), :], out_sem)

    in_copy(0).start()
    in_copy(1).start()

    def chunk_body(ch, _):
        in_copy(ch).wait()

        @pl.when(ch >= 2)
        def _():
            out_copy(ch - 2).wait()

        ib = inbuf.at[lax.rem(ch, 2)]
        ob = outbuf.at[lax.rem(ch, 2)]

        @plsc.parallel_loop(0, CH, unroll=2)
        def body(r, ib=ib, ob=ob, colv=colv):
            rows = jnp.full((16,), r, jnp.int32)
            for g in range(K // 16):
                ob[r, pl.ds(g * 16, 16)] = plsc.load_gather(ib, [rows, colv[g]])
        out_copy(ch).start()

        @pl.when(ch + 2 < NCH)
        def _():
            in_copy(ch + 2).start()

        return 0

    lax.fori_loop(0, NCH, chunk_body, 0, unroll=1)
    out_copy(NCH - 2).wait()
    out_copy(NCH - 1).wait()


def kernel(inputs):
    return _sc_extract(inputs)


# speedup vs baseline: 1.5447x; 1.0004x over previous
import functools

import jax
import jax.numpy as jnp
from jax import lax
from jax.experimental import pallas as pl
from jax.experimental.pallas import tpu as pltpu
from jax.experimental.pallas import tpu_sc as plsc

R, C, K = 16384, 512, 128
NC, NS = 2, 16
NW = NC * NS
ROWS_PER_W = R // NW
CH = 64
NCH = ROWS_PER_W // CH

_mesh = plsc.VectorSubcoreMesh(core_axis_name="c", subcore_axis_name="s")


@functools.partial(
    pl.kernel,
    mesh=_mesh,
    out_type=jax.ShapeDtypeStruct((R, K), jnp.float32),
    scratch_types=[
        pltpu.VMEM((3, CH, C), jnp.float32),
        pltpu.VMEM((2, CH, K), jnp.float32),
        pltpu.SemaphoreType.DMA,
        pltpu.SemaphoreType.DMA,
    ],
    compiler_params=pltpu.CompilerParams(needs_layout_passes=False),
)
def _sc_extract(in_hbm, out_hbm, inbuf, outbuf, in_sem, out_sem):
    wid = lax.axis_index("s") * NC + lax.axis_index("c")
    lane = lax.iota(jnp.int32, 16)
    colv = [lane * 4 + 64 * g for g in range(K // 16)]
    base = wid * ROWS_PER_W

    def in_copy(ch):
        return pltpu.make_async_copy(
            in_hbm.at[pl.ds(base + ch * CH, CH), :],
            inbuf.at[lax.rem(ch, 3)], in_sem)

    def out_copy(ch):
        return pltpu.make_async_copy(
            outbuf.at[lax.rem(ch, 2)],
            out_hbm.at[pl.ds(base + ch * CH, CH), :], out_sem)

    in_copy(0).start()
    in_copy(1).start()
    in_copy(2).start()

    def chunk_body(ch, _):
        in_copy(ch).wait()

        @pl.when(ch >= 2)
        def _():
            out_copy(ch - 2).wait()

        ib = inbuf.at[lax.rem(ch, 3)]
        ob = outbuf.at[lax.rem(ch, 2)]

        @plsc.parallel_loop(0, CH, unroll=2)
        def body(r, ib=ib, ob=ob, colv=colv):
            rows = jnp.full((16,), r, jnp.int32)
            for g in range(K // 16):
                ob[r, pl.ds(g * 16, 16)] = plsc.load_gather(ib, [rows, colv[g]])
        out_copy(ch).start()

        @pl.when(ch + 3 < NCH)
        def _():
            in_copy(ch + 3).start()

        return 0

    lax.fori_loop(0, NCH, chunk_body, 0, unroll=1)
    out_copy(NCH - 2).wait()
    out_copy(NCH - 1).wait()


def kernel(inputs):
    return _sc_extract(inputs)
